# baseline (speedup 1.0000x reference)
import functools

import jax
import jax.numpy as jnp
from jax import lax
from jax.experimental import pallas as pl
from jax.experimental.pallas import tpu as pltpu
from jax.experimental.pallas import tpu_sc as plsc

_B, _T, _C = 4, 1024, 512
_G, _V = 2, 320
_GV = _G * _V
_D = 128
_N = _B * _T
_BLK = 512
_GRID = _N // _BLK
_MAX_TEMP = 2.0

_NC, _NS = 2, 16
_NW = _NC * _NS
_ROWS = _N * _G
_RPW = _ROWS // _NW
_CHUNK = 128
_NCHUNK = _RPW // _CHUNK


def _proj_kernel(x_ref, wt_ref, b_ref, idx_ref, ppl_ref, acc_ref):
    i = pl.program_id(0)

    @pl.when(i == 0)
    def _init():
        acc_ref[...] = jnp.zeros_like(acc_ref)

    logits = jnp.dot(x_ref[...], wt_ref[...],
                     preferred_element_type=jnp.float32) + b_ref[...]

    cols = jax.lax.broadcasted_iota(jnp.int32, (_BLK, _GV), 1)
    g0 = cols < _V
    neg = jnp.float32(-jnp.inf)
    l0 = jnp.where(g0, logits, neg)
    l1 = jnp.where(g0, neg, logits)
    m0 = jnp.max(l0, axis=1, keepdims=True)
    m1 = jnp.max(l1, axis=1, keepdims=True)
    idx0 = jnp.min(jnp.where(l0 == m0, cols, _GV), axis=1, keepdims=True)
    idx1 = jnp.min(jnp.where(l1 == m1, cols, _GV), axis=1, keepdims=True)
    idx_ref[...] = jnp.concatenate([idx0, idx1], axis=1)

    m = jnp.where(g0, m0, m1)
    e = jnp.exp(logits - m)
    s0 = jnp.sum(jnp.where(g0, e, 0.0), axis=1, keepdims=True)
    s1 = jnp.sum(jnp.where(g0, 0.0, e), axis=1, keepdims=True)
    probs = e / jnp.where(g0, s0, s1)
    acc_ref[...] += jnp.sum(probs, axis=0, keepdims=True)

    @pl.when(i == _GRID - 1)
    def _epilogue():
        avg = acc_ref[...] / jnp.float32(_N)
        plogp = avg * jnp.log(avg + jnp.float32(1e-7))
        c1 = jax.lax.broadcasted_iota(jnp.int32, (1, _GV), 1)
        in_g0 = c1 < _V
        ent0 = -jnp.sum(jnp.where(in_g0, plogp, 0.0))
        ent1 = -jnp.sum(jnp.where(in_g0, 0.0, plogp))
        ppl = jnp.exp(ent0) + jnp.exp(ent1)
        val = (jnp.float32(_GV) - ppl) / jnp.float32(_GV)
        ppl_ref[...] = jnp.full((1, 1), val, jnp.float32)


@functools.partial(
    pl.kernel,
    out_type=jax.ShapeDtypeStruct((_ROWS, _D), jnp.float32),
    mesh=plsc.VectorSubcoreMesh(core_axis_name="c", subcore_axis_name="s",
                                num_cores=_NC, num_subcores=_NS),
    scratch_types=[
        pltpu.VMEM((_CHUNK,), jnp.int32),
        pltpu.VMEM((_CHUNK, _D), jnp.float32),
        pltpu.SemaphoreType.DMA,
    ],
)
def _gather_kernel(table_hbm, idx_hbm, out_hbm, idx_v, rows_v, sem):
    wid = lax.axis_index("s") * _NC + lax.axis_index("c")
    base = wid * _RPW
    for c in range(_NCHUNK):
        off = base + c * _CHUNK
        pltpu.sync_copy(idx_hbm.at[pl.ds(off, _CHUNK)], idx_v)
        pltpu.async_copy(table_hbm.at[idx_v], rows_v, sem).wait()
        pltpu.sync_copy(rows_v, out_hbm.at[pl.ds(off, _CHUNK)])


def kernel(x, W, b, codebook):
    flat = x.reshape(_N, _C)
    wt = W.T
    b2 = b.reshape(1, _GV)
    table = codebook.reshape(_GV, _D)

    idx, ppl = pl.pallas_call(
        _proj_kernel,
        grid=(_GRID,),
        in_specs=[
            pl.BlockSpec((_BLK, _C), lambda i: (i, 0)),
            pl.BlockSpec((_C, _GV), lambda i: (0, 0)),
            pl.BlockSpec((1, _GV), lambda i: (0, 0)),
        ],
        out_specs=[
            pl.BlockSpec((_BLK, _G), lambda i: (i, 0)),
            pl.BlockSpec((1, 1), lambda i: (0, 0)),
        ],
        out_shape=[
            jax.ShapeDtypeStruct((_N, _G), jnp.int32),
            jax.ShapeDtypeStruct((1, 1), jnp.float32),
        ],
        scratch_shapes=[pltpu.VMEM((1, _GV), jnp.float32)],
    )(flat, wt, b2)

    rows = _gather_kernel(table, idx.reshape(_ROWS))
    out = rows.reshape(_B, _T, _G * _D)
    return (out, ppl.reshape(()), jnp.float32(_MAX_TEMP))


# speedup vs baseline: 1.1789x; 1.1789x over previous
import functools

import jax
import jax.numpy as jnp
from jax import lax
from jax.experimental import pallas as pl
from jax.experimental.pallas import tpu as pltpu
from jax.experimental.pallas import tpu_sc as plsc

_B, _T, _C = 4, 1024, 512
_G, _V = 2, 320
_GV = _G * _V
_D = 128
_N = _B * _T
_BLK = 512
_GRID = _N // _BLK
_MAX_TEMP = 2.0

_NC, _NS = 2, 16
_NW = _NC * _NS
_CHUNK = 128
_TPW = _N // (_NW // _G)


def _proj_kernel(x_ref, wt_ref, b_ref, idx_ref, ppl_ref, acc_ref):
    i = pl.program_id(0)

    @pl.when(i == 0)
    def _init():
        acc_ref[...] = jnp.zeros_like(acc_ref)

    logits = jnp.dot(x_ref[...], wt_ref[...],
                     preferred_element_type=jnp.float32) + b_ref[...]

    cols = jax.lax.broadcasted_iota(jnp.int32, (_BLK, _GV), 1)
    g0 = cols < _V
    neg = jnp.float32(-jnp.inf)
    l0 = jnp.where(g0, logits, neg)
    l1 = jnp.where(g0, neg, logits)
    m0 = jnp.max(l0, axis=1, keepdims=True)
    m1 = jnp.max(l1, axis=1, keepdims=True)
    idx0 = jnp.min(jnp.where(l0 == m0, cols, _GV), axis=1, keepdims=True)
    idx1 = jnp.min(jnp.where(l1 == m1, cols, _GV), axis=1, keepdims=True)
    idx_ref[...] = jnp.concatenate([idx0, idx1], axis=1).T

    m = jnp.where(g0, m0, m1)
    e = jnp.exp(logits - m)
    s0 = jnp.sum(jnp.where(g0, e, 0.0), axis=1, keepdims=True)
    s1 = jnp.sum(jnp.where(g0, 0.0, e), axis=1, keepdims=True)
    probs = e / jnp.where(g0, s0, s1)
    acc_ref[...] += jnp.sum(probs, axis=0, keepdims=True)

    @pl.when(i == _GRID - 1)
    def _epilogue():
        avg = acc_ref[...] / jnp.float32(_N)
        plogp = avg * jnp.log(avg + jnp.float32(1e-7))
        c1 = jax.lax.broadcasted_iota(jnp.int32, (1, _GV), 1)
        in_g0 = c1 < _V
        ent0 = -jnp.sum(jnp.where(in_g0, plogp, 0.0))
        ent1 = -jnp.sum(jnp.where(in_g0, 0.0, plogp))
        ppl = jnp.exp(ent0) + jnp.exp(ent1)
        val = (jnp.float32(_GV) - ppl) / jnp.float32(_GV)
        ppl_ref[...] = jnp.full((1, 1), val, jnp.float32)


@functools.partial(
    pl.kernel,
    out_type=jax.ShapeDtypeStruct((_N, _G * _D), jnp.float32),
    mesh=plsc.VectorSubcoreMesh(core_axis_name="c", subcore_axis_name="s",
                                num_cores=_NC, num_subcores=_NS),
    scratch_types=[
        pltpu.VMEM((_CHUNK,), jnp.int32),
        pltpu.VMEM((_CHUNK, _D), jnp.float32),
        pltpu.SemaphoreType.DMA,
    ],
)
def _gather_kernel(table_hbm, idx_hbm, out_hbm, idx_v, rows_v, sem):
    wid = lax.axis_index("s") * _NC + lax.axis_index("c")
    g = wid % _G
    chunk = wid // _G
    t0 = chunk * _TPW
    for c in range(_TPW // _CHUNK):
        toff = t0 + c * _CHUNK
        pltpu.sync_copy(idx_hbm.at[g, pl.ds(toff, _CHUNK)], idx_v)
        pltpu.async_copy(table_hbm.at[idx_v], rows_v, sem).wait()
        pltpu.sync_copy(rows_v, out_hbm.at[pl.ds(toff, _CHUNK),
                                           pl.ds(g * _D, _D)])


def kernel(x, W, b, codebook):
    flat = x.reshape(_N, _C)
    wt = W.T
    b2 = b.reshape(1, _GV)
    table = codebook.reshape(_GV, _D)

    idx, ppl = pl.pallas_call(
        _proj_kernel,
        grid=(_GRID,),
        in_specs=[
            pl.BlockSpec((_BLK, _C), lambda i: (i, 0)),
            pl.BlockSpec((_C, _GV), lambda i: (0, 0)),
            pl.BlockSpec((1, _GV), lambda i: (0, 0)),
        ],
        out_specs=[
            pl.BlockSpec((_G, _BLK), lambda i: (0, i)),
            pl.BlockSpec((1, 1), lambda i: (0, 0)),
        ],
        out_shape=[
            jax.ShapeDtypeStruct((_G, _N), jnp.int32),
            jax.ShapeDtypeStruct((1, 1), jnp.float32),
        ],
        scratch_shapes=[pltpu.VMEM((1, _GV), jnp.float32)],
    )(flat, wt, b2)

    rows = _gather_kernel(table, idx)
    out = rows.reshape(_B, _T, _G * _D)
    return (out, ppl.reshape(()), jnp.float32(_MAX_TEMP))


# speedup vs baseline: 1.2394x; 1.0514x over previous
import functools

import jax
import jax.numpy as jnp
from jax import lax
from jax.experimental import pallas as pl
from jax.experimental.pallas import tpu as pltpu
from jax.experimental.pallas import tpu_sc as plsc

_B, _T, _C = 4, 1024, 512
_G, _V = 2, 320
_GV = _G * _V
_D = 128
_N = _B * _T
_BLK = 512
_GRID = _N // _BLK
_MAX_TEMP = 2.0

_NC, _NS = 2, 16
_NW = _NC * _NS
_CHUNK = 128
_TPW = _N // (_NW // _G)


def _proj_kernel(x_ref, w_ref, b_ref, idx_ref, ppl_ref, acc_ref):
    i = pl.program_id(0)

    @pl.when(i == 0)
    def _init():
        acc_ref[...] = jnp.zeros_like(acc_ref)

    logits = lax.dot_general(x_ref[0], w_ref[...],
                             (((1,), (1,)), ((), ())),
                             preferred_element_type=jnp.float32) + b_ref[...]

    cols = jax.lax.broadcasted_iota(jnp.int32, (_BLK, _GV), 1)
    g0 = cols < _V
    neg = jnp.float32(-jnp.inf)
    l0 = jnp.where(g0, logits, neg)
    l1 = jnp.where(g0, neg, logits)
    m0 = jnp.max(l0, axis=1, keepdims=True)
    m1 = jnp.max(l1, axis=1, keepdims=True)
    idx0 = jnp.min(jnp.where(l0 == m0, cols, _GV), axis=1, keepdims=True)
    idx1 = jnp.min(jnp.where(l1 == m1, cols, _GV), axis=1, keepdims=True)
    idx_ref[...] = jnp.concatenate([idx0, idx1], axis=1).T

    m = jnp.where(g0, m0, m1)
    e = jnp.exp(logits - m)
    s0 = jnp.sum(jnp.where(g0, e, 0.0), axis=1, keepdims=True)
    s1 = jnp.sum(jnp.where(g0, 0.0, e), axis=1, keepdims=True)
    r = jnp.where(g0, 1.0 / s0, 1.0 / s1)
    acc_ref[...] += jnp.sum(e * r, axis=0, keepdims=True)

    @pl.when(i == _GRID - 1)
    def _epilogue():
        avg = acc_ref[...] / jnp.float32(_N)
        plogp = avg * jnp.log(avg + jnp.float32(1e-7))
        c1 = jax.lax.broadcasted_iota(jnp.int32, (1, _GV), 1)
        in_g0 = c1 < _V
        ent0 = -jnp.sum(jnp.where(in_g0, plogp, 0.0))
        ent1 = -jnp.sum(jnp.where(in_g0, 0.0, plogp))
        ppl = jnp.exp(ent0) + jnp.exp(ent1)
        val = (jnp.float32(_GV) - ppl) / jnp.float32(_GV)
        ppl_ref[...] = jnp.full((1, 1), val, jnp.float32)


@functools.partial(
    pl.kernel,
    out_type=jax.ShapeDtypeStruct((_B, _T, _G * _D), jnp.float32),
    mesh=plsc.VectorSubcoreMesh(core_axis_name="c", subcore_axis_name="s",
                                num_cores=_NC, num_subcores=_NS),
    scratch_types=[
        pltpu.VMEM((_CHUNK,), jnp.int32),
        pltpu.VMEM((_CHUNK,), jnp.int32),
        pltpu.VMEM((_CHUNK, _D), jnp.float32),
        pltpu.VMEM((_CHUNK, _D), jnp.float32),
        pltpu.SemaphoreType.DMA,
        pltpu.SemaphoreType.DMA,
        pltpu.SemaphoreType.DMA,
        pltpu.SemaphoreType.DMA,
    ],
)
def _gather_kernel(table_hbm, idx_hbm, out_hbm,
                   idx_v0, idx_v1, rows_v0, rows_v1, g0s, g1s, w0s, w1s):
    wid = lax.axis_index("s") * _NC + lax.axis_index("c")
    g = wid % _G
    chunk = wid // _G
    t0 = chunk * _TPW
    b = t0 // _T
    trem = t0 - b * _T
    pltpu.sync_copy(idx_hbm.at[g, pl.ds(t0, _CHUNK)], idx_v0)
    pltpu.sync_copy(idx_hbm.at[g, pl.ds(t0 + _CHUNK, _CHUNK)], idx_v1)
    cp0 = pltpu.async_copy(table_hbm.at[idx_v0], rows_v0, g0s)
    cp1 = pltpu.async_copy(table_hbm.at[idx_v1], rows_v1, g1s)
    cp0.wait()
    wb0 = pltpu.async_copy(
        rows_v0, out_hbm.at[b, pl.ds(trem, _CHUNK), pl.ds(g * _D, _D)], w0s)
    cp1.wait()
    wb1 = pltpu.async_copy(
        rows_v1, out_hbm.at[b, pl.ds(trem + _CHUNK, _CHUNK),
                            pl.ds(g * _D, _D)], w1s)
    wb0.wait()
    wb1.wait()


def kernel(x, W, b, codebook):
    b2 = b.reshape(1, _GV)
    table = codebook.reshape(_GV, _D)
    tpb = _T // _BLK

    idx, ppl = pl.pallas_call(
        _proj_kernel,
        grid=(_GRID,),
        in_specs=[
            pl.BlockSpec((1, _BLK, _C), lambda i: (i // tpb, i % tpb, 0)),
            pl.BlockSpec((_GV, _C), lambda i: (0, 0)),
            pl.BlockSpec((1, _GV), lambda i: (0, 0)),
        ],
        out_specs=[
            pl.BlockSpec((_G, _BLK), lambda i: (0, i)),
            pl.BlockSpec((1, 1), lambda i: (0, 0)),
        ],
        out_shape=[
            jax.ShapeDtypeStruct((_G, _N), jnp.int32),
            jax.ShapeDtypeStruct((1, 1), jnp.float32),
        ],
        scratch_shapes=[pltpu.VMEM((1, _GV), jnp.float32)],
    )(x, W, b2)

    out = _gather_kernel(table, idx)
    return (out, ppl.reshape(()), jnp.float32(_MAX_TEMP))
